# baseline (speedup 1.0000x reference)
import functools

import jax
import jax.numpy as jnp
from jax.experimental import pallas as pl
from jax.experimental.pallas import tpu as pltpu

_MiB = 1024 * 1024


def _pipelined_body(x_hbm, w_hbm, b_ref, o_hbm,
                    x_vmem, w_vmem, y_ref, rd_sems, w_sem, out_sems,
                    *, n_tiles, tm, d, n_slots):
    def rd_copy(i):
        sl = pl.ds(i * tm, tm)
        return pltpu.make_async_copy(x_hbm.at[sl, :], x_vmem.at[sl, :],
                                     rd_sems.at[i])

    def w_copy():
        return pltpu.make_async_copy(w_hbm, w_vmem, w_sem)

    def out_copy(i):
        slot = i % n_slots
        return pltpu.make_async_copy(y_ref.at[slot],
                                     o_hbm.at[pl.ds(i * tm, tm), :],
                                     out_sems.at[slot])

    w_copy().start()
    for i in range(n_tiles):
        rd_copy(i).start()
    w_copy().wait()

    for i in range(n_tiles):
        rd_copy(i).wait()
        if i >= n_slots:
            out_copy(i - n_slots).wait()
        slot = i % n_slots
        xt = x_vmem[pl.ds(i * tm, tm), :]
        y = jax.lax.dot_general(xt, w_vmem[...],
                                dimension_numbers=(((1,), (1,)), ((), ())),
                                preferred_element_type=jnp.float32)
        y = y + b_ref[...]
        y_ref[slot, :, :d] = y
        y_ref[slot, :, d:] = y
        out_copy(i).start()

    for i in range(max(0, n_tiles - n_slots), n_tiles):
        out_copy(i).wait()


def kernel(x, weight, bias):
    B, S, D = x.shape
    rows = B * S
    scale = 2

    tm = 8
    for cand in (512, 256, 128, 64, 32, 16, 8):
        if rows % cand == 0:
            tm = cand
            break
    n_tiles = rows // tm
    n_slots = min(3, n_tiles)

    x2d = x.reshape(rows, D)
    b2d = bias.astype(jnp.float32).reshape(1, D)

    body = functools.partial(_pipelined_body, n_tiles=n_tiles, tm=tm, d=D,
                             n_slots=n_slots)
    out2d = pl.pallas_call(
        body,
        out_shape=jax.ShapeDtypeStruct((rows, scale * D), x.dtype),
        in_specs=[
            pl.BlockSpec(memory_space=pl.ANY),
            pl.BlockSpec(memory_space=pl.ANY),
            pl.BlockSpec(memory_space=pltpu.VMEM),
        ],
        out_specs=pl.BlockSpec(memory_space=pl.ANY),
        scratch_shapes=[
            pltpu.VMEM((rows, D), jnp.float32),
            pltpu.VMEM((D, D), jnp.float32),
            pltpu.VMEM((n_slots, tm, scale * D), jnp.float32),
            pltpu.SemaphoreType.DMA((n_tiles,)),
            pltpu.SemaphoreType.DMA(()),
            pltpu.SemaphoreType.DMA((n_slots,)),
        ],
        compiler_params=pltpu.CompilerParams(
            vmem_limit_bytes=56 * _MiB,
        ),
    )(x2d, weight, b2d)

    return out2d.reshape(rows, scale, D).reshape(B, S * scale, D)


# speedup vs baseline: 1.0361x; 1.0361x over previous
import functools

import jax
import jax.numpy as jnp
from jax.experimental import pallas as pl
from jax.experimental.pallas import tpu as pltpu

_MiB = 1024 * 1024


def _pipelined_body(x_hbm, w_vmem, b_ref, o_hbm,
                    x_vmem, y_ref, rd_sems, out_sems,
                    *, n_tiles, tm, d, n_slots):
    def rd_copy(i):
        sl = pl.ds(i * tm, tm)
        return pltpu.make_async_copy(x_hbm.at[sl, :], x_vmem.at[sl, :],
                                     rd_sems.at[i])

    def out_copy(i):
        slot = i % n_slots
        return pltpu.make_async_copy(y_ref.at[slot],
                                     o_hbm.at[pl.ds(i * tm, tm), :],
                                     out_sems.at[slot])

    for i in range(n_tiles):
        rd_copy(i).start()

    for i in range(n_tiles):
        rd_copy(i).wait()
        if i >= n_slots:
            out_copy(i - n_slots).wait()
        slot = i % n_slots
        xt = x_vmem[pl.ds(i * tm, tm), :]
        y = jax.lax.dot_general(xt, w_vmem[...],
                                dimension_numbers=(((1,), (1,)), ((), ())),
                                preferred_element_type=jnp.float32)
        y = y + b_ref[...]
        y_ref[slot, :, :d] = y
        y_ref[slot, :, d:] = y
        out_copy(i).start()

    for i in range(max(0, n_tiles - n_slots), n_tiles):
        out_copy(i).wait()


def kernel(x, weight, bias):
    B, S, D = x.shape
    rows = B * S
    scale = 2

    tm = 8
    for cand in (512, 256, 128, 64, 32, 16, 8):
        if rows % cand == 0:
            tm = cand
            break
    n_tiles = rows // tm
    n_slots = min(4, n_tiles)

    x2d = x.reshape(rows, D)
    b2d = bias.astype(jnp.float32).reshape(1, D)

    body = functools.partial(_pipelined_body, n_tiles=n_tiles, tm=tm, d=D,
                             n_slots=n_slots)
    out2d = pl.pallas_call(
        body,
        out_shape=jax.ShapeDtypeStruct((rows, scale * D), x.dtype),
        in_specs=[
            pl.BlockSpec(memory_space=pl.ANY),
            pl.BlockSpec(memory_space=pltpu.VMEM),
            pl.BlockSpec(memory_space=pltpu.VMEM),
        ],
        out_specs=pl.BlockSpec(memory_space=pl.ANY),
        scratch_shapes=[
            pltpu.VMEM((rows, D), jnp.float32),
            pltpu.VMEM((n_slots, tm, scale * D), jnp.float32),
            pltpu.SemaphoreType.DMA((n_tiles,)),
            pltpu.SemaphoreType.DMA((n_slots,)),
        ],
        compiler_params=pltpu.CompilerParams(
            vmem_limit_bytes=56 * _MiB,
        ),
    )(x2d, weight, b2d)

    return out2d.reshape(rows, scale, D).reshape(B, S * scale, D)


# speedup vs baseline: 1.0383x; 1.0021x over previous
import functools

import jax
import jax.numpy as jnp
from jax.experimental import pallas as pl
from jax.experimental.pallas import tpu as pltpu

_MiB = 1024 * 1024


def _pipelined_body(x_hbm, w_vmem, b_ref, o_hbm,
                    x_vmem, y_ref, rd_sems, out_sems,
                    *, schedule, n_rd, rd_chunk, d, n_slots, slot_rows):
    def rd_copy(i):
        sl = pl.ds(i * rd_chunk, rd_chunk)
        return pltpu.make_async_copy(x_hbm.at[sl, :], x_vmem.at[sl, :],
                                     rd_sems.at[i])

    for i in range(n_rd):
        rd_copy(i).start()

    rd_done = 0
    started = []
    for j, (r0, nr) in enumerate(schedule):
        need_chunk = (r0 + nr - 1) // rd_chunk
        while rd_done <= need_chunk:
            rd_copy(rd_done).wait()
            rd_done += 1
        if j >= n_slots:
            started[j - n_slots].wait()
        slot = j % n_slots
        xt = x_vmem[pl.ds(r0, nr), :]
        y = jax.lax.dot_general(xt, w_vmem[...],
                                dimension_numbers=(((1,), (1,)), ((), ())),
                                preferred_element_type=jnp.float32)
        y = y + b_ref[...]
        y_ref[slot, pl.ds(0, nr), pl.ds(0, d)] = y
        y_ref[slot, pl.ds(0, nr), pl.ds(d, d)] = y
        desc = pltpu.make_async_copy(y_ref.at[slot, pl.ds(0, nr)],
                                     o_hbm.at[pl.ds(r0, nr), :],
                                     out_sems.at[slot])
        desc.start()
        started.append(desc)

    for desc in started[-n_slots:]:
        desc.wait()


def kernel(x, weight, bias):
    B, S, D = x.shape
    rows = B * S
    scale = 2

    rd_chunk = 8
    for cand in (512, 256, 128, 64, 32, 16, 8):
        if rows % cand == 0:
            rd_chunk = cand
            break
    n_rd = rows // rd_chunk

    schedule = []
    r0 = 0
    if rows % 512 == 0 and rows >= 1024:
        while r0 < 512:
            schedule.append((r0, 128))
            r0 += 128
    while r0 < rows:
        nr = min(rd_chunk, rows - r0)
        schedule.append((r0, nr))
        r0 += nr
    slot_rows = max(nr for _, nr in schedule)
    n_slots = min(4, len(schedule))

    x2d = x.reshape(rows, D)
    b2d = bias.astype(jnp.float32).reshape(1, D)

    body = functools.partial(_pipelined_body, schedule=tuple(schedule),
                             n_rd=n_rd, rd_chunk=rd_chunk, d=D,
                             n_slots=n_slots, slot_rows=slot_rows)
    out2d = pl.pallas_call(
        body,
        out_shape=jax.ShapeDtypeStruct((rows, scale * D), x.dtype),
        in_specs=[
            pl.BlockSpec(memory_space=pl.ANY),
            pl.BlockSpec(memory_space=pltpu.VMEM),
            pl.BlockSpec(memory_space=pltpu.VMEM),
        ],
        out_specs=pl.BlockSpec(memory_space=pl.ANY),
        scratch_shapes=[
            pltpu.VMEM((rows, D), jnp.float32),
            pltpu.VMEM((n_slots, slot_rows, scale * D), jnp.float32),
            pltpu.SemaphoreType.DMA((n_rd,)),
            pltpu.SemaphoreType.DMA((n_slots,)),
        ],
        compiler_params=pltpu.CompilerParams(
            vmem_limit_bytes=56 * _MiB,
        ),
    )(x2d, weight, b2d)

    return out2d.reshape(rows, scale, D).reshape(B, S * scale, D)


# speedup vs baseline: 1.0384x; 1.0001x over previous
import functools

import jax
import jax.numpy as jnp
from jax.experimental import pallas as pl
from jax.experimental.pallas import tpu as pltpu

_MiB = 1024 * 1024


def _pipelined_body(x_hbm, w_vmem, b_ref, o_hbm,
                    x_vmem, y_ref, rd_sems, out_sems,
                    *, schedule, rd_sched, d, n_slots, slot_rows):
    def rd_copy(i):
        c0, cn = rd_sched[i]
        sl = pl.ds(c0, cn)
        return pltpu.make_async_copy(x_hbm.at[sl, :], x_vmem.at[sl, :],
                                     rd_sems.at[i])

    for i in range(len(rd_sched)):
        rd_copy(i).start()

    rd_done = 0
    started = []
    for j, (r0, nr) in enumerate(schedule):
        while rd_done < len(rd_sched) and rd_sched[rd_done][0] < r0 + nr:
            rd_copy(rd_done).wait()
            rd_done += 1
        if j >= n_slots:
            started[j - n_slots].wait()
        slot = j % n_slots
        xt = x_vmem[pl.ds(r0, nr), :]
        y = jax.lax.dot_general(xt, w_vmem[...],
                                dimension_numbers=(((1,), (1,)), ((), ())),
                                preferred_element_type=jnp.float32)
        y = y + b_ref[...]
        y_ref[slot, pl.ds(0, nr), pl.ds(0, d)] = y
        y_ref[slot, pl.ds(0, nr), pl.ds(d, d)] = y
        desc = pltpu.make_async_copy(y_ref.at[slot, pl.ds(0, nr)],
                                     o_hbm.at[pl.ds(r0, nr), :],
                                     out_sems.at[slot])
        desc.start()
        started.append(desc)

    for desc in started[-n_slots:]:
        desc.wait()


def kernel(x, weight, bias):
    B, S, D = x.shape
    rows = B * S
    scale = 2

    rd_chunk = 8
    for cand in (512, 256, 128, 64, 32, 16, 8):
        if rows % cand == 0:
            rd_chunk = cand
            break
    rd_sched = []
    c0 = 0
    while c0 < rows:
        cn = rd_chunk if c0 == 0 else min(4 * rd_chunk, rows - c0)
        rd_sched.append((c0, cn))
        c0 += cn
    n_rd = len(rd_sched)

    schedule = []
    r0 = 0
    if rows % 512 == 0 and rows >= 1024:
        while r0 < 512:
            schedule.append((r0, 128))
            r0 += 128
    while r0 < rows:
        nr = min(rd_chunk, rows - r0)
        schedule.append((r0, nr))
        r0 += nr
    slot_rows = max(nr for _, nr in schedule)
    n_slots = min(4, len(schedule))

    x2d = x.reshape(rows, D)
    b2d = bias.astype(jnp.float32).reshape(1, D)

    body = functools.partial(_pipelined_body, schedule=tuple(schedule),
                             rd_sched=tuple(rd_sched), d=D,
                             n_slots=n_slots, slot_rows=slot_rows)
    out2d = pl.pallas_call(
        body,
        out_shape=jax.ShapeDtypeStruct((rows, scale * D), x.dtype),
        in_specs=[
            pl.BlockSpec(memory_space=pl.ANY),
            pl.BlockSpec(memory_space=pltpu.VMEM),
            pl.BlockSpec(memory_space=pltpu.VMEM),
        ],
        out_specs=pl.BlockSpec(memory_space=pl.ANY),
        scratch_shapes=[
            pltpu.VMEM((rows, D), jnp.float32),
            pltpu.VMEM((n_slots, slot_rows, scale * D), jnp.float32),
            pltpu.SemaphoreType.DMA((n_rd,)),
            pltpu.SemaphoreType.DMA((n_slots,)),
        ],
        compiler_params=pltpu.CompilerParams(
            vmem_limit_bytes=56 * _MiB,
        ),
    )(x2d, weight, b2d)

    return out2d.reshape(rows, scale, D).reshape(B, S * scale, D)
